# baseline (speedup 1.0000x reference)
import functools

import jax
import jax.numpy as jnp
from jax import lax
from jax.experimental import pallas as pl
from jax.experimental.pallas import tpu as pltpu
from jax.experimental.pallas import tpu_sc as plsc

VOCAB = 8352
EMBED_DIM = 3
SEQ = 200
BATCH = 16384
NUM_WORKERS = 32
BCOLS_PER_W = BATCH // NUM_WORKERS
BLK_T = 8
BLK_B = 512
N_TROWS = SEQ // BLK_T
BLKS_PER_ROW = BCOLS_PER_W // BLK_B
N_BLKS = N_TROWS * BLKS_PER_ROW
GROUPS_PER_TROW = BLK_B // 16
EP_START = N_BLKS - 2 if N_BLKS % 2 == 0 else N_BLKS - 3
N_STEADY_PAIRS = (EP_START - 2) // 2

_mesh = plsc.VectorSubcoreMesh(core_axis_name="c", subcore_axis_name="s")


@functools.partial(
    pl.kernel,
    out_type=jax.ShapeDtypeStruct((EMBED_DIM, SEQ, BATCH), jnp.float32),
    mesh=_mesh,
    compiler_params=pltpu.CompilerParams(needs_layout_passes=False),
    scratch_types=[
        pltpu.VMEM((VOCAB,), jnp.float32),
        pltpu.VMEM((VOCAB,), jnp.float32),
        pltpu.VMEM((VOCAB,), jnp.float32),
        pltpu.VMEM((BLK_T, BLK_B), jnp.int32),
        pltpu.VMEM((BLK_T, BLK_B), jnp.int32),
        pltpu.VMEM((EMBED_DIM, BLK_T, BLK_B), jnp.float32),
        pltpu.VMEM((EMBED_DIM, BLK_T, BLK_B), jnp.float32),
        pltpu.SemaphoreType.DMA,
        pltpu.SemaphoreType.DMA,
        pltpu.SemaphoreType.DMA,
        pltpu.SemaphoreType.DMA,
        pltpu.SemaphoreType.DMA,
    ],
)
def _embed_sc(x_ref, tbl_ref, out_ref, t0c, t1c, t2c,
              idx0, idx1, out0, out1, si0, si1, so0, so1, stbl):
    wid = lax.axis_index("s") * 2 + lax.axis_index("c")
    wb = wid * BCOLS_PER_W

    idx_b = (idx0, idx1)
    out_b = (out0, out1)
    s_in = (si0, si1)
    s_out = (so0, so1)
    tcols = (t0c, t1c, t2c)

    def blk_pos(blk):
        trow = blk // BLKS_PER_ROW
        half = blk % BLKS_PER_ROW
        return trow * BLK_T, wb + half * BLK_B

    def start_in(b, blk):
        tt, bb = blk_pos(blk)
        pltpu.make_async_copy(
            x_ref.at[pl.ds(tt, BLK_T), pl.ds(bb, BLK_B)], idx_b[b], s_in[b]
        ).start()

    def wait_in(b):
        pltpu.make_async_copy(
            x_ref.at[pl.ds(0, BLK_T), pl.ds(0, BLK_B)], idx_b[b], s_in[b]
        ).wait()

    def start_out(b, blk):
        tt, bb = blk_pos(blk)
        pltpu.make_async_copy(
            out_b[b], out_ref.at[:, pl.ds(tt, BLK_T), pl.ds(bb, BLK_B)], s_out[b]
        ).start()

    def wait_out(b):
        pltpu.make_async_copy(
            out_b[b], out_ref.at[:, pl.ds(0, BLK_T), pl.ds(0, BLK_B)], s_out[b]
        ).wait()

    def compute(b):
        @plsc.parallel_loop(0, BLK_T, unroll=4)
        def _(t):
            for j in range(GROUPS_PER_TROW):
                c = j * 16
                idx = idx_b[b][t, pl.ds(c, 16)]
                for d in range(EMBED_DIM):
                    out_b[b][d, t, pl.ds(c, 16)] = plsc.load_gather(
                        tcols[d], [idx]
                    )

    for d in range(EMBED_DIM):
        pltpu.make_async_copy(
            tbl_ref.at[pl.ds(d * VOCAB, VOCAB)], tcols[d], stbl
        ).start()
    start_in(0, 0)
    start_in(1, 1)
    for d in range(EMBED_DIM):
        pltpu.make_async_copy(
            tbl_ref.at[pl.ds(0, VOCAB)], tcols[d], stbl
        ).wait()

    for b in range(2):
        wait_in(b)
        compute(b)
        start_out(b, b)
        start_in(b, b + 2)

    @pl.loop(0, N_STEADY_PAIRS)
    def _(i):
        for b in range(2):
            blk = 2 + i * 2 + b
            wait_in(b)
            wait_out(b)
            compute(b)
            start_out(b, blk)
            start_in(b, blk + 2)

    for blk in range(2 + 2 * N_STEADY_PAIRS, N_BLKS):
        b = blk % 2
        wait_in(b)
        wait_out(b)
        compute(b)
        start_out(b, blk)
        if blk + 2 < N_BLKS:
            start_in(b, blk + 2)
    wait_out(0)
    wait_out(1)


def kernel(x, table):
    xt = x.astype(jnp.int32).T
    tbl_t = table.T.reshape(EMBED_DIM * VOCAB)
    out = _embed_sc(xt, tbl_t)
    return out.transpose(2, 1, 0)


# speedup vs baseline: 1.2458x; 1.2458x over previous
import functools

import jax
import jax.numpy as jnp
from jax import lax
from jax.experimental import pallas as pl
from jax.experimental.pallas import tpu as pltpu
from jax.experimental.pallas import tpu_sc as plsc

VOCAB = 8352
EMBED_DIM = 3
SEQ = 200
BATCH = 16384
NUM_WORKERS = 32
BCOLS_PER_W = BATCH // NUM_WORKERS
BLK_T = 8
BLK_B = 256
N_TROWS = SEQ // BLK_T
BLKS_PER_ROW = BCOLS_PER_W // BLK_B
N_BLKS = N_TROWS * BLKS_PER_ROW
GROUPS_PER_TROW = BLK_B // 16
EP_START = N_BLKS - 2 if N_BLKS % 2 == 0 else N_BLKS - 3
N_STEADY_PAIRS = (EP_START - 2) // 2

_mesh = plsc.VectorSubcoreMesh(core_axis_name="c", subcore_axis_name="s")


@functools.partial(
    pl.kernel,
    out_type=jax.ShapeDtypeStruct((EMBED_DIM, SEQ, BATCH), jnp.float32),
    mesh=_mesh,
    compiler_params=pltpu.CompilerParams(needs_layout_passes=False),
    scratch_types=[
        pltpu.VMEM((VOCAB,), jnp.float32),
        pltpu.VMEM((VOCAB,), jnp.float32),
        pltpu.VMEM((VOCAB,), jnp.float32),
        pltpu.VMEM((BLK_T, BLK_B), jnp.int32),
        pltpu.VMEM((BLK_T, BLK_B), jnp.int32),
        pltpu.VMEM((EMBED_DIM, BLK_T, BLK_B), jnp.float32),
        pltpu.VMEM((EMBED_DIM, BLK_T, BLK_B), jnp.float32),
        pltpu.SemaphoreType.DMA,
        pltpu.SemaphoreType.DMA,
        pltpu.SemaphoreType.DMA,
        pltpu.SemaphoreType.DMA,
        pltpu.SemaphoreType.DMA,
    ],
)
def _embed_sc(x_ref, tbl_ref, out_ref, t0c, t1c, t2c,
              idx0, idx1, out0, out1, si0, si1, so0, so1, stbl):
    wid = lax.axis_index("s") * 2 + lax.axis_index("c")
    wb = wid * BCOLS_PER_W

    idx_b = (idx0, idx1)
    out_b = (out0, out1)
    s_in = (si0, si1)
    s_out = (so0, so1)
    tcols = (t0c, t1c, t2c)

    def blk_pos(blk):
        trow = blk // BLKS_PER_ROW
        half = blk % BLKS_PER_ROW
        return trow * BLK_T, wb + half * BLK_B

    def start_in(b, blk):
        tt, bb = blk_pos(blk)
        pltpu.make_async_copy(
            x_ref.at[pl.ds(tt, BLK_T), pl.ds(bb, BLK_B)], idx_b[b], s_in[b]
        ).start()

    def wait_in(b):
        pltpu.make_async_copy(
            x_ref.at[pl.ds(0, BLK_T), pl.ds(0, BLK_B)], idx_b[b], s_in[b]
        ).wait()

    def start_out(b, blk):
        tt, bb = blk_pos(blk)
        pltpu.make_async_copy(
            out_b[b], out_ref.at[:, pl.ds(tt, BLK_T), pl.ds(bb, BLK_B)], s_out[b]
        ).start()

    def wait_out(b):
        pltpu.make_async_copy(
            out_b[b], out_ref.at[:, pl.ds(0, BLK_T), pl.ds(0, BLK_B)], s_out[b]
        ).wait()

    def compute(b):
        @plsc.parallel_loop(0, BLK_T, unroll=4)
        def _(t):
            for j in range(GROUPS_PER_TROW):
                c = j * 16
                idx = idx_b[b][t, pl.ds(c, 16)]
                for d in range(EMBED_DIM):
                    out_b[b][d, t, pl.ds(c, 16)] = plsc.load_gather(
                        tcols[d], [idx]
                    )

    for d in range(EMBED_DIM):
        pltpu.make_async_copy(
            tbl_ref.at[pl.ds(d * VOCAB, VOCAB)], tcols[d], stbl
        ).start()
    start_in(0, 0)
    start_in(1, 1)
    for d in range(EMBED_DIM):
        pltpu.make_async_copy(
            tbl_ref.at[pl.ds(0, VOCAB)], tcols[d], stbl
        ).wait()

    for b in range(2):
        wait_in(b)
        compute(b)
        start_out(b, b)
        start_in(b, b + 2)

    @pl.loop(0, N_STEADY_PAIRS)
    def _(i):
        for b in range(2):
            blk = 2 + i * 2 + b
            wait_in(b)
            wait_out(b)
            compute(b)
            start_out(b, blk)
            start_in(b, blk + 2)

    for blk in range(2 + 2 * N_STEADY_PAIRS, N_BLKS):
        b = blk % 2
        wait_in(b)
        wait_out(b)
        compute(b)
        start_out(b, blk)
        if blk + 2 < N_BLKS:
            start_in(b, blk + 2)
    wait_out(0)
    wait_out(1)


def kernel(x, table):
    xt = x.astype(jnp.int32).T
    tbl_t = table.T.reshape(EMBED_DIM * VOCAB)
    out = _embed_sc(xt, tbl_t)
    return out.transpose(2, 1, 0)
